# baseline (device time: 106213 ns/iter reference)
import os

import jax
import jax.numpy as jnp
from jax import lax
from jax.experimental import pallas as pl
from jax.experimental.pallas import tpu as pltpu

N_DEV = 8
N_TOK = 2048
D_IN = 512
D_OUT = 1024
N_EXP = 32
E_LOCAL = N_EXP // N_DEV
CHUNK = N_TOK // N_DEV

if os.environ.get("KPARTS", "3") == "1":
    PARTS = ((0, 1024),)
    DIMS = ((1, 3, 4),)
else:
    PARTS = ((0, 384), (384, 384), (768, 256))
    DIMS = ((1, 3, 4), (3, 4, 1), (4, 1, 3))

RS_BASE = (0, 4, 6)
AG_BASE = (0, 1, 3)


_KMODE = os.environ.get("KMODE", "full")


def _xor_span(dims):
    s = {0}
    for d in dims:
        s |= {v ^ d for v in s}
    return sorted(s)


def kernel(x, router_W, route_idx, expert_W, shared_W):
    def body(x_ref, rw_ref, idx_ref, ew_ref, sw_ref, out_ref,
             red_ref, rs_buf, rs_send, rs_recv, ag_send, ag_recv):
        pos = lax.axis_index("i")

        scores = jnp.dot(x_ref[:, :], rw_ref[:, :],
                         preferred_element_type=jnp.float32)
        s_max = jnp.max(scores, axis=1, keepdims=True)
        ex = jnp.exp(scores - s_max)
        probs = ex / jnp.sum(ex, axis=1, keepdims=True)
        idx = idx_ref[:, :]
        cols = lax.broadcasted_iota(jnp.int32, (N_TOK, N_EXP), 1)
        p_sel = jnp.sum(jnp.where(cols == idx, probs, 0.0), axis=1,
                        keepdims=True)

        if _KMODE != "compute_only":
            barrier = pltpu.get_barrier_semaphore()
            for g in (1, 3, 4):
                pl.semaphore_signal(barrier, inc=1, device_id=(pos ^ g,),
                                    device_id_type=pl.DeviceIdType.MESH)
            pl.semaphore_wait(barrier, 3)

        def start_round(p, off, w, r, bases, sends, recvs):
            g = DIMS[p][r]
            partner = pos ^ g
            started = []
            for i, j in enumerate(_xor_span(DIMS[p][r + 1:])):
                slot = bases[r] + i
                sem = p * 7 + slot
                src_c = partner ^ j
                d = pltpu.make_async_remote_copy(
                    src_ref=red_ref.at[pl.ds(src_c * CHUNK, CHUNK),
                                       pl.ds(off, w)],
                    dst_ref=rs_buf.at[slot, :, pl.ds(off, w)],
                    send_sem=sends.at[sem],
                    recv_sem=recvs.at[sem],
                    device_id=(partner,),
                    device_id_type=pl.DeviceIdType.MESH,
                )
                d.start()
                started.append((p, off, w, j, slot, d))
            return started

        x_bf = x_ref[:, :].astype(jnp.bfloat16)
        coefs = [jnp.where(idx == pos * E_LOCAL + j, p_sel, 0.0)
                 for j in range(E_LOCAL)]
        r0_started = []
        for p, (off, w) in enumerate(PARTS):
            if _KMODE == "comm_only":
                red_ref[:, off:off + w] = jnp.zeros((N_TOK, w), jnp.bfloat16)
            else:
                accp = jnp.zeros((N_TOK, w), jnp.float32)
                for j in range(E_LOCAL):
                    wj = ew_ref[j, :, off:off + w].astype(jnp.bfloat16)
                    accp = accp + coefs[j] * jnp.dot(
                        x_bf, wj, preferred_element_type=jnp.float32)
                red_ref[:, off:off + w] = accp.astype(jnp.bfloat16)
            if _KMODE != "compute_only":
                r0_started.extend(start_round(p, off, w, 0, RS_BASE,
                                              rs_send, rs_recv))

        if _KMODE == "comm_only":
            out_ref[:, :] = jnp.zeros((N_TOK, D_OUT), jnp.float32)
        else:
            sw_bf = sw_ref[:, :].astype(jnp.bfloat16)
            out_ref[:, :] = jnp.dot(x_bf, sw_bf,
                                    preferred_element_type=jnp.float32)

        for r in range(3 if _KMODE != "compute_only" else 0):
            if r == 0:
                started = r0_started
            else:
                started = []
                for p, (off, w) in enumerate(PARTS):
                    started.extend(start_round(p, off, w, r, RS_BASE,
                                               rs_send, rs_recv))
            for _, _, _, _, _, d in started:
                d.wait()
            for p, off, w, j, slot, _ in started:
                rs = pl.ds((pos ^ j) * CHUNK, CHUNK)
                cs = pl.ds(off, w)
                red_ref[rs, cs] = red_ref[rs, cs] + rs_buf[slot, :, cs]

        for k in range(3 if _KMODE != "compute_only" else 0):
            started = []
            for p, (off, w) in enumerate(PARTS):
                g = DIMS[p][2 - k]
                partner = pos ^ g
                for i, j in enumerate(_xor_span(DIMS[p][3 - k:])):
                    sem = p * 7 + AG_BASE[k] + i
                    sl = (pl.ds((pos ^ j) * CHUNK, CHUNK), pl.ds(off, w))
                    d = pltpu.make_async_remote_copy(
                        src_ref=red_ref.at[sl],
                        dst_ref=red_ref.at[sl],
                        send_sem=ag_send.at[sem],
                        recv_sem=ag_recv.at[sem],
                        device_id=(partner,),
                        device_id_type=pl.DeviceIdType.MESH,
                    )
                    d.start()
                    started.append(d)
            for d in started:
                d.wait()

        out_ref[:, :] = out_ref[:, :] + red_ref[:, :].astype(jnp.float32)

    return pl.pallas_call(
        body,
        out_shape=jax.ShapeDtypeStruct((N_TOK, D_OUT), jnp.float32),
        in_specs=[pl.BlockSpec(memory_space=pltpu.VMEM)] * 5,
        out_specs=pl.BlockSpec(memory_space=pltpu.VMEM),
        scratch_shapes=[
            pltpu.VMEM((N_TOK, D_OUT), jnp.bfloat16),
            pltpu.VMEM((7, CHUNK, D_OUT), jnp.bfloat16),
            pltpu.SemaphoreType.DMA((21,)),
            pltpu.SemaphoreType.DMA((21,)),
            pltpu.SemaphoreType.DMA((21,)),
            pltpu.SemaphoreType.DMA((21,)),
        ],
        compiler_params=pltpu.CompilerParams(collective_id=0),
    )(x, router_W, route_idx, expert_W, shared_W)


# device time: 68656 ns/iter; 1.5470x vs baseline; 1.5470x over previous
import os

import jax
import jax.numpy as jnp
from jax import lax
from jax.experimental import pallas as pl
from jax.experimental.pallas import tpu as pltpu

N_DEV = 8
N_TOK = 2048
D_IN = 512
D_OUT = 1024
N_EXP = 32
E_LOCAL = N_EXP // N_DEV
CHUNK = N_TOK // N_DEV

PARTS = ((0, 384), (384, 384), (768, 256))
DIMS = ((1, 3, 4), (3, 4, 1), (4, 1, 3))

RS_BASE = (0, 4, 6)
AG_BASE = (0, 1, 3)

_KMODE = os.environ.get("KMODE", "full")


def _xor_span(dims):
    s = {0}
    for d in dims:
        s |= {v ^ d for v in s}
    return sorted(s)


def kernel(x, router_W, route_idx, expert_W, shared_W):
    def body(x_ref, rw_ref, idx_ref, ew_ref, sw_ref, out_ref,
             red0, red1, red2, buf0, buf1, buf2,
             rs_send, rs_recv, ag_send, ag_recv):
        reds = (red0, red1, red2)
        bufs = (buf0, buf1, buf2)
        pos = lax.axis_index("i")

        scores = jnp.dot(x_ref[:, :], rw_ref[:, :],
                         preferred_element_type=jnp.float32)
        s_max = jnp.max(scores, axis=1, keepdims=True)
        ex = jnp.exp(scores - s_max)
        probs = ex / jnp.sum(ex, axis=1, keepdims=True)
        idx = idx_ref[:, :]
        cols = lax.broadcasted_iota(jnp.int32, (N_TOK, N_EXP), 1)
        p_sel = jnp.sum(jnp.where(cols == idx, probs, 0.0), axis=1,
                        keepdims=True)

        if _KMODE != "compute_only":
            barrier = pltpu.get_barrier_semaphore()
            for g in (1, 3, 4):
                pl.semaphore_signal(barrier, inc=1, device_id=(pos ^ g,),
                                    device_id_type=pl.DeviceIdType.MESH)
            pl.semaphore_wait(barrier, 3)

        def make_rs(p, r):
            g = DIMS[p][r]
            partner = pos ^ g
            started = []
            for i, j in enumerate(_xor_span(DIMS[p][r + 1:])):
                slot = RS_BASE[r] + i
                sem = p * 7 + slot
                d = pltpu.make_async_remote_copy(
                    src_ref=reds[p].at[pl.ds(((partner ^ j) * CHUNK), CHUNK), :],
                    dst_ref=bufs[p].at[slot],
                    send_sem=rs_send.at[sem],
                    recv_sem=rs_recv.at[sem],
                    device_id=(partner,),
                    device_id_type=pl.DeviceIdType.MESH,
                )
                d.start()
                started.append((j, slot, d))
            return started

        def finish_rs(p, started):
            for _, _, d in started:
                d.wait()
            for j, slot, _ in started:
                rs = pl.ds((pos ^ j) * CHUNK, CHUNK)
                reds[p][rs, :] = reds[p][rs, :] + bufs[p][slot, :, :]

        def make_ag(p, k):
            g = DIMS[p][2 - k]
            partner = pos ^ g
            started = []
            for i, j in enumerate(_xor_span(DIMS[p][3 - k:])):
                sem = p * 7 + AG_BASE[k] + i
                sl = pl.ds((pos ^ j) * CHUNK, CHUNK)
                d = pltpu.make_async_remote_copy(
                    src_ref=reds[p].at[sl, :],
                    dst_ref=reds[p].at[sl, :],
                    send_sem=ag_send.at[sem],
                    recv_sem=ag_recv.at[sem],
                    device_id=(partner,),
                    device_id_type=pl.DeviceIdType.MESH,
                )
                d.start()
                started.append(d)
            return started

        x_bf = x_ref[:, :].astype(jnp.bfloat16)
        coefs = [jnp.where(idx == pos * E_LOCAL + j, p_sel, 0.0)
                 for j in range(E_LOCAL)]
        inflight = [None, None, None]
        for p, (off, w) in enumerate(PARTS):
            if _KMODE == "comm_only":
                reds[p][:, :] = jnp.zeros((N_TOK, w), jnp.bfloat16)
            else:
                accp = jnp.zeros((N_TOK, w), jnp.float32)
                for j in range(E_LOCAL):
                    wj = ew_ref[j, :, off:off + w].astype(jnp.bfloat16)
                    accp = accp + coefs[j] * jnp.dot(
                        x_bf, wj, preferred_element_type=jnp.float32)
                reds[p][:, :] = accp.astype(jnp.bfloat16)
            if _KMODE != "compute_only":
                inflight[p] = make_rs(p, 0)

        if _KMODE == "comm_only":
            out_ref[:, :] = jnp.zeros((N_TOK, D_OUT), jnp.float32)
        else:
            sw_bf = sw_ref[:, :].astype(jnp.bfloat16)
            out_ref[:, :] = jnp.dot(x_bf, sw_bf,
                                    preferred_element_type=jnp.float32)

        if _KMODE != "compute_only":
            for r in (1, 2):
                for p in range(3):
                    finish_rs(p, inflight[p])
                    inflight[p] = make_rs(p, r)
            for p in range(3):
                finish_rs(p, inflight[p])
                inflight[p] = make_ag(p, 0)
            for k in (1, 2):
                for p in range(3):
                    for d in inflight[p]:
                        d.wait()
                    inflight[p] = make_ag(p, k)

            for p, (off, w) in enumerate(PARTS):
                for d in inflight[p]:
                    d.wait()
                out_ref[:, off:off + w] = (out_ref[:, off:off + w]
                                           + reds[p][:, :].astype(jnp.float32))

    return pl.pallas_call(
        body,
        out_shape=jax.ShapeDtypeStruct((N_TOK, D_OUT), jnp.float32),
        in_specs=[pl.BlockSpec(memory_space=pltpu.VMEM)] * 5,
        out_specs=pl.BlockSpec(memory_space=pltpu.VMEM),
        scratch_shapes=[
            pltpu.VMEM((N_TOK, PARTS[0][1]), jnp.bfloat16),
            pltpu.VMEM((N_TOK, PARTS[1][1]), jnp.bfloat16),
            pltpu.VMEM((N_TOK, PARTS[2][1]), jnp.bfloat16),
            pltpu.VMEM((7, CHUNK, PARTS[0][1]), jnp.bfloat16),
            pltpu.VMEM((7, CHUNK, PARTS[1][1]), jnp.bfloat16),
            pltpu.VMEM((7, CHUNK, PARTS[2][1]), jnp.bfloat16),
            pltpu.SemaphoreType.DMA((21,)),
            pltpu.SemaphoreType.DMA((21,)),
            pltpu.SemaphoreType.DMA((21,)),
            pltpu.SemaphoreType.DMA((21,)),
        ],
        compiler_params=pltpu.CompilerParams(collective_id=0),
    )(x, router_W, route_idx, expert_W, shared_W)


# device time: 63399 ns/iter; 1.6753x vs baseline; 1.0829x over previous
import os

import jax
import jax.numpy as jnp
from jax import lax
from jax.experimental import pallas as pl
from jax.experimental.pallas import tpu as pltpu

N_DEV = 8
N_TOK = 2048
D_IN = 512
D_OUT = 1024
N_EXP = 32
E_LOCAL = N_EXP // N_DEV
CHUNK = N_TOK // N_DEV

_WIDTHS = (128, 256, 128, 256, 128, 128)
_OFFS = tuple(sum(_WIDTHS[:i]) for i in range(len(_WIDTHS)))
PARTS = tuple(zip(_OFFS, _WIDTHS))
_DIM_ROT = ((1, 3, 4), (3, 4, 1), (4, 1, 3))
if os.environ.get("KDIMS", "rot") == "same":
    DIMS = tuple(_DIM_ROT[0] for p in range(len(PARTS)))
else:
    DIMS = tuple(_DIM_ROT[p % 3] for p in range(len(PARTS)))
NP = len(PARTS)

RS_BASE = (0, 4, 6)
AG_BASE = (0, 1, 3)

_KMODE = os.environ.get("KMODE", "full")


def _xor_span(dims):
    s = {0}
    for d in dims:
        s |= {v ^ d for v in s}
    return sorted(s)


def kernel(x, router_W, route_idx, expert_W, shared_W):
    def body(x_ref, rw_ref, idx_ref, ew_ref, sw_ref, out_ref, *scr):
        reds = scr[:NP]
        bufs = scr[NP:2 * NP]
        rs_send, rs_recv, ag_send, ag_recv = scr[2 * NP:]
        pos = lax.axis_index("i")

        scores = jnp.dot(x_ref[:, :], rw_ref[:, :],
                         preferred_element_type=jnp.float32)
        s_max = jnp.max(scores, axis=1, keepdims=True)
        ex = jnp.exp(scores - s_max)
        probs = ex / jnp.sum(ex, axis=1, keepdims=True)
        idx = idx_ref[:, :]
        cols = lax.broadcasted_iota(jnp.int32, (N_TOK, N_EXP), 1)
        p_sel = jnp.sum(jnp.where(cols == idx, probs, 0.0), axis=1,
                        keepdims=True)

        if _KMODE != "compute_only":
            barrier = pltpu.get_barrier_semaphore()
            for g in (1, 3, 4):
                pl.semaphore_signal(barrier, inc=1, device_id=(pos ^ g,),
                                    device_id_type=pl.DeviceIdType.MESH)
            pl.semaphore_wait(barrier, 3)

        def make_rs(p, r):
            g = DIMS[p][r]
            partner = pos ^ g
            started = []
            for i, j in enumerate(_xor_span(DIMS[p][r + 1:])):
                slot = RS_BASE[r] + i
                sem = p * 7 + slot
                d = pltpu.make_async_remote_copy(
                    src_ref=reds[p].at[pl.ds(((partner ^ j) * CHUNK), CHUNK), :],
                    dst_ref=bufs[p].at[slot],
                    send_sem=rs_send.at[sem],
                    recv_sem=rs_recv.at[sem],
                    device_id=(partner,),
                    device_id_type=pl.DeviceIdType.MESH,
                )
                d.start()
                started.append((j, slot, d))
            return started

        def finish_rs(p, started):
            for _, _, d in started:
                d.wait()
            for j, slot, _ in started:
                rs = pl.ds((pos ^ j) * CHUNK, CHUNK)
                reds[p][rs, :] = reds[p][rs, :] + bufs[p][slot, :, :]

        def make_ag(p, k):
            g = DIMS[p][2 - k]
            partner = pos ^ g
            started = []
            for i, j in enumerate(_xor_span(DIMS[p][3 - k:])):
                sem = p * 7 + AG_BASE[k] + i
                sl = pl.ds((pos ^ j) * CHUNK, CHUNK)
                d = pltpu.make_async_remote_copy(
                    src_ref=reds[p].at[sl, :],
                    dst_ref=reds[p].at[sl, :],
                    send_sem=ag_send.at[sem],
                    recv_sem=ag_recv.at[sem],
                    device_id=(partner,),
                    device_id_type=pl.DeviceIdType.MESH,
                )
                d.start()
                started.append(d)
            return started

        x_bf = x_ref[:, :].astype(jnp.bfloat16)
        coefs = [jnp.where(idx == pos * E_LOCAL + j, p_sel, 0.0)
                 for j in range(E_LOCAL)]
        inflight = [None] * NP
        for p, (off, w) in enumerate(PARTS):
            if _KMODE == "comm_only":
                reds[p][:, :] = jnp.zeros((N_TOK, w), jnp.bfloat16)
            else:
                accp = jnp.zeros((N_TOK, w), jnp.float32)
                for j in range(E_LOCAL):
                    wj = ew_ref[j, :, off:off + w].astype(jnp.bfloat16)
                    accp = accp + coefs[j] * jnp.dot(
                        x_bf, wj, preferred_element_type=jnp.float32)
                reds[p][:, :] = accp.astype(jnp.bfloat16)
            if _KMODE != "compute_only":
                inflight[p] = make_rs(p, 0)

        if _KMODE == "comm_only":
            out_ref[:, :] = jnp.zeros((N_TOK, D_OUT), jnp.float32)
        else:
            sw_bf = sw_ref[:, :].astype(jnp.bfloat16)
            out_ref[:, :] = jnp.dot(x_bf, sw_bf,
                                    preferred_element_type=jnp.float32)

        def combine(p, chunk_ids):
            off, w = PARTS[p]
            for c in chunk_ids:
                rs = pl.ds(c * CHUNK, CHUNK)
                out_ref[rs, off:off + w] = (
                    out_ref[rs, off:off + w]
                    + reds[p][rs, :].astype(jnp.float32))

        def ag_recv_ids(p, m):
            g = DIMS[p][2 - m]
            return [pos ^ g ^ j for j in _xor_span(DIMS[p][3 - m:])]

        if _KMODE != "compute_only":
            for r in (1, 2):
                for p in range(NP):
                    finish_rs(p, inflight[p])
                    inflight[p] = make_rs(p, r)
            for p in range(NP):
                finish_rs(p, inflight[p])
                inflight[p] = make_ag(p, 0)
                combine(p, [pos])
            for k in (1, 2):
                for p in range(NP):
                    for d in inflight[p]:
                        d.wait()
                    inflight[p] = make_ag(p, k)
                    combine(p, ag_recv_ids(p, k - 1))
            for p in range(NP):
                for d in inflight[p]:
                    d.wait()
                combine(p, ag_recv_ids(p, 2))

    return pl.pallas_call(
        body,
        out_shape=jax.ShapeDtypeStruct((N_TOK, D_OUT), jnp.float32),
        in_specs=[pl.BlockSpec(memory_space=pltpu.VMEM)] * 5,
        out_specs=pl.BlockSpec(memory_space=pltpu.VMEM),
        scratch_shapes=(
            [pltpu.VMEM((N_TOK, w), jnp.bfloat16) for _, w in PARTS]
            + [pltpu.VMEM((7, CHUNK, w), jnp.bfloat16) for _, w in PARTS]
            + [pltpu.SemaphoreType.DMA((NP * 7,)) for _ in range(4)]
        ),
        compiler_params=pltpu.CompilerParams(collective_id=0),
    )(x, router_W, route_idx, expert_W, shared_W)
